# baseline (speedup 1.0000x reference)
import functools

import jax
import jax.numpy as jnp
from jax import lax
from jax.experimental import pallas as pl
from jax.experimental.pallas import tpu as pltpu
from jax.experimental.pallas import tpu_sc as plsc

_N = 15
_M = 64
_SWARM = 16
_MAX_COV = 10.0
_L = 16
_NC = 2
_NS = 16
_NW = _NC * _NS
_CR = 32


def _bf16r(x):
    u = plsc.bitcast(x, jnp.uint32)
    u = (u + jnp.uint32(0x7FFF)
         + (lax.shift_right_logical(u, jnp.uint32(16)) & jnp.uint32(1))) \
        & jnp.uint32(0xFFFF0000)
    return plsc.bitcast(u, jnp.float32)


def _sc_body(rpw, feat_hbm, cam_hbm, qb_hbm, cond_hbm,
             prob_hbm, pos_hbm, cov_hbm, idx_hbm,
             feat_v, cam_v, qb_v, prob_v, pos_v, cov_v, idxo_v, cond_v):
    wid = lax.axis_index("c") * _NS + lax.axis_index("s")
    io = lax.iota(jnp.int32, _L)
    m15 = io < _N

    pltpu.sync_copy(cond_hbm, cond_v)

    def row_body(r, carry):
        cb = r * (_M * 3)
        kv = [[plsc.load_gather(cam_v, [cb + 48 * j + c + 3 * io])
               for c in range(3)] for j in range(4)]
        maskf = []
        kb = []
        for j in range(4):
            sq = kv[j][0] * kv[j][0] + kv[j][1] * kv[j][1] + kv[j][2] * kv[j][2]
            maskf.append(jnp.where(sq < 1e-8, -jnp.inf, 0.0).astype(jnp.float32))
            kb.append([_bf16r(kv[j][c]) for c in range(3)])
        fb = r * (_N * 8)
        pr = [plsc.load_gather(feat_v, [fb + 8 * io + c], mask=m15)
              for c in range(3)]
        disk = plsc.load_gather(feat_v, [fb + 8 * io + 7], mask=m15)
        qbb = r * (_N * 3)
        qxv = plsc.load_gather(qb_v, [qbb + 3 * io], mask=m15)
        qyv = plsc.load_gather(qb_v, [qbb + 3 * io + 1], mask=m15)
        qzv = plsc.load_gather(qb_v, [qbb + 3 * io + 2], mask=m15)

        kk = lax.rem(r, _SWARM)
        condb = plsc.load_gather(cond_v, [jnp.broadcast_to(kk, (_L,))]) != 0

        rmv = jnp.zeros((_L,), jnp.float32)
        idxv = jnp.zeros((_L,), jnp.int32)
        pb = r * (_N * _M)
        for n in range(_N):
            qx, qy, qz = qxv[n], qyv[n], qzv[n]
            s = []
            for j in range(4):
                p0 = qx * kb[j][0]
                p1 = qy * kb[j][1]
                p2 = qz * kb[j][2]
                s1 = p0 + p1
                bp = s1 - p0
                e1 = (p0 - (s1 - bp)) + (p1 - bp)
                s2 = s1 + p2
                cp = s2 - s1
                e2 = (s1 - (s2 - cp)) + (p2 - cp)
                v = e1 + e2
                vp = v - e1
                w = (e1 - (v - vp)) + (e2 - vp)
                u = plsc.bitcast(v, jnp.int32)
                sgn = (u ^ plsc.bitcast(w, jnp.int32)) >= 0
                du = jnp.where(sgn, u + 1, u - 1)
                even = (u & 1) == 0
                ur = jnp.where((w != 0.0) & even, du, u)
                vr = plsc.bitcast(ur, jnp.float32)
                s.append((s2 + vr) + maskf[j])
            rm = jnp.max(jnp.maximum(jnp.maximum(s[0], s[1]),
                                     jnp.maximum(s[2], s[3])))
            e = [jnp.exp(s[j] - rm) for j in range(4)]
            ssum = jnp.sum((e[0] + e[1]) + (e[2] + e[3]))
            inv = jnp.ones((_L,), jnp.float32) / jnp.broadcast_to(ssum, (_L,))
            for j in range(4):
                prob_v[pl.ds(pb + n * _M + _L * j, _L)] = jnp.where(
                    condb, jnp.float32(0.0), e[j] * inv)
            thr = rm - (jnp.abs(rm) * jnp.float32(5e-7) + jnp.float32(1e-30))
            cand = [jnp.where(s[j] >= thr, io + _L * j, jnp.int32(1000))
                    for j in range(4)]
            am = jnp.min(jnp.minimum(jnp.minimum(cand[0], cand[1]),
                                     jnp.minimum(cand[2], cand[3])))
            lane = io == n
            rmv = jnp.where(lane, rm, rmv)
            idxv = jnp.where(lane, am, idxv)

        validc = (rmv > 0.99) & jnp.logical_not(condb)
        covv = jnp.clip((1.0 - rmv) * 100.0, 0.01, _MAX_COV)
        covv = jnp.where(validc, covv, jnp.float32(_MAX_COV))
        idxf = jnp.where(validc, idxv.astype(jnp.float32), jnp.float32(-1.0))
        gbase = cb + 3 * idxv
        for c in range(3):
            cg = plsc.load_gather(cam_v, [gbase + c], mask=m15)
            posc = jnp.where(validc, disk * cg, pr[c])
            plsc.store_scatter(pos_v, [r * 45 + 3 * io + c], posc, mask=m15)
        plsc.store_scatter(cov_v, [r * _N + io], covv, mask=m15)
        plsc.store_scatter(idxo_v, [r * _N + io], idxf, mask=m15)
        return carry

    def chunk_body(ci, carry):
        base = wid * rpw + ci * _CR
        pltpu.sync_copy(feat_hbm.at[pl.ds(base * 120, _CR * 120)],
                        feat_v.at[pl.ds(0, _CR * 120)])
        pltpu.sync_copy(cam_hbm.at[pl.ds(base * 192, _CR * 192)], cam_v)
        pltpu.sync_copy(qb_hbm.at[pl.ds(base * 45, _CR * 45)],
                        qb_v.at[pl.ds(0, _CR * 45)])
        lax.fori_loop(0, _CR, row_body, 0)
        pltpu.sync_copy(prob_v, prob_hbm.at[pl.ds(base * 960, _CR * 960)])
        pltpu.sync_copy(pos_v, pos_hbm.at[pl.ds(base * 45, _CR * 45)])
        pltpu.sync_copy(cov_v, cov_hbm.at[pl.ds(base * 15, _CR * 15)])
        pltpu.sync_copy(idxo_v, idx_hbm.at[pl.ds(base * 15, _CR * 15)])
        return carry

    lax.fori_loop(0, rpw // _CR, chunk_body, 0)


def kernel(others_feat, others_cam):
    bsn = others_feat.shape[0] // _N
    bs = bsn // _SWARM
    rpw = bsn // _NW
    f32 = jnp.float32

    cam3 = others_cam.reshape(bsn, _M, 3)
    lost = jnp.linalg.norm(cam3, ord=2, axis=2) < 1e-4
    cond_a = jnp.all(lost.reshape(bs, _SWARM, _M), axis=(0, 2))
    disk_all = others_feat.reshape(bsn, _N, 8)[:, :, 7]
    cond_b = jnp.any(disk_all.reshape(bs, _SWARM, _N) < 1e-4, axis=(0, 2))
    cond = (cond_a | cond_b).astype(jnp.int32)

    feat_flat = others_feat.reshape(-1)
    cam_flat = others_cam.reshape(-1)

    pos3 = others_feat.reshape(bsn * _N, 8)[:, :3]
    nrm = jnp.linalg.norm(pos3, ord=2, axis=-1, keepdims=True)
    qn = pos3 / jnp.maximum(nrm, 1e-12)
    qb_flat = qn.astype(jnp.bfloat16).astype(f32).reshape(-1)

    mesh = plsc.VectorSubcoreMesh(core_axis_name="c", subcore_axis_name="s",
                                  num_cores=_NC, num_subcores=_NS)
    fn = pl.kernel(
        functools.partial(_sc_body, rpw),
        out_type=(
            jax.ShapeDtypeStruct((bsn * _N * _M,), f32),
            jax.ShapeDtypeStruct((bsn * _N * 3,), f32),
            jax.ShapeDtypeStruct((bsn * _N,), f32),
            jax.ShapeDtypeStruct((bsn * _N,), f32),
        ),
        mesh=mesh,
        compiler_params=pltpu.CompilerParams(needs_layout_passes=False),
        scratch_types=[
            pltpu.VMEM((_CR * 120 + _L,), f32),
            pltpu.VMEM((_CR * 192,), f32),
            pltpu.VMEM((_CR * 45 + _L,), f32),
            pltpu.VMEM((_CR * 960,), f32),
            pltpu.VMEM((_CR * 45,), f32),
            pltpu.VMEM((_CR * 15,), f32),
            pltpu.VMEM((_CR * 15,), f32),
            pltpu.VMEM((_SWARM,), jnp.int32),
        ],
    )
    probf, posf, covf, idxf = fn(feat_flat, cam_flat, qb_flat, cond)

    out_prob = probf.reshape(bsn, _N, _M)
    out_pos = posf.reshape(bsn, _N, 3)
    out_cov = covf.reshape(bsn, _N, 1)
    out_idx = idxf.reshape(bsn, _N, 1)
    out_scores = jnp.full((bsn, _SWARM, _M + 1), -jnp.inf, f32)
    return out_prob, out_pos, out_cov, out_scores, out_idx
